# Initial kernel scaffold; baseline (speedup 1.0000x reference)
import jax
import jax.numpy as jnp
from jax.experimental import pallas as pl


def kernel(hs, rs, ts, ent_embs, rel_embs, ent_transfer, rel_transfer):
    raise NotImplementedError("write your pallas kernel here")


# speedup vs baseline: 2.5078x; 2.5078x over previous
import functools

import jax
import jax.numpy as jnp
from jax import lax
from jax.experimental import pallas as pl
from jax.experimental.pallas import tpu as pltpu
from jax.experimental.pallas import tpu_sc as plsc

DIM = 128
MARGIN = 2.0
NC, NS = 2, 16
NW = NC * NS
CHUNK = 64
LANES = 16
KREG = DIM // LANES


def _allsum(v, perms):
    for p in perms:
        v = v + v.at[p].get(mode="promise_in_bounds")
    return v


def _rsqrt_f32(x):
    xb = lax.bitcast_convert_type(x, jnp.int32)
    yb = jnp.int32(0x5F3759DF) - lax.shift_right_logical(xb, 1)
    y = lax.bitcast_convert_type(yb, jnp.float32)
    hx = x * jnp.float32(0.5)
    for _ in range(3):
        y = y * (jnp.float32(1.5) - hx * y * y)
    return y


def _make_sc_kernel(B):
    assert B % (NW * CHUNK) == 0
    per_w = B // NW
    n_chunks = per_w // CHUNK
    mesh = plsc.VectorSubcoreMesh(core_axis_name="c", subcore_axis_name="s")

    @functools.partial(
        pl.kernel,
        mesh=mesh,
        out_type=jax.ShapeDtypeStruct((B,), jnp.float32),
        scratch_types=[
            pltpu.VMEM((CHUNK,), jnp.int32),
            pltpu.VMEM((CHUNK,), jnp.int32),
            pltpu.VMEM((CHUNK,), jnp.int32),
            pltpu.VMEM((CHUNK, DIM), jnp.float32),
            pltpu.VMEM((CHUNK, DIM), jnp.float32),
            pltpu.VMEM((CHUNK, DIM), jnp.float32),
            pltpu.VMEM((CHUNK, DIM), jnp.float32),
            pltpu.VMEM((CHUNK, DIM), jnp.float32),
            pltpu.VMEM((CHUNK, DIM), jnp.float32),
            pltpu.VMEM((per_w,), jnp.float32),
            pltpu.SemaphoreType.DMA,
        ],
    )
    def sc_kernel(hs_h, rs_h, ts_h, ee_h, re_h, et_h, rt_h, out_h,
                  hidx, ridx, tidx, hbuf, tbuf, rbuf, htrb, ttrb, rtrb,
                  outv, sem):
        wid = lax.axis_index("s") * NC + lax.axis_index("c")
        base_w = wid * per_w
        lane_iota = lax.iota(jnp.int32, LANES)
        perms = [lax.bitwise_xor(lane_iota, jnp.int32(s)) for s in (8, 4, 2, 1)]

        def chunk_body(ci, _):
            base = base_w + ci * CHUNK
            pltpu.sync_copy(hs_h.at[pl.ds(base, CHUNK)], hidx)
            pltpu.sync_copy(rs_h.at[pl.ds(base, CHUNK)], ridx)
            pltpu.sync_copy(ts_h.at[pl.ds(base, CHUNK)], tidx)
            cps = [
                pltpu.async_copy(ee_h.at[hidx], hbuf, sem),
                pltpu.async_copy(ee_h.at[tidx], tbuf, sem),
                pltpu.async_copy(re_h.at[ridx], rbuf, sem),
                pltpu.async_copy(et_h.at[hidx], htrb, sem),
                pltpu.async_copy(et_h.at[tidx], ttrb, sem),
                pltpu.async_copy(rt_h.at[ridx], rtrb, sem),
            ]
            for cp in cps:
                cp.wait()

            def tri_score(i):
                acc_h = jnp.zeros((LANES,), jnp.float32)
                acc_t = jnp.zeros((LANES,), jnp.float32)
                hk, tk = [], []
                for k in range(KREG):
                    h = hbuf[i, pl.ds(k * LANES, LANES)]
                    ht = htrb[i, pl.ds(k * LANES, LANES)]
                    t = tbuf[i, pl.ds(k * LANES, LANES)]
                    tt = ttrb[i, pl.ds(k * LANES, LANES)]
                    acc_h = acc_h + h * ht
                    acc_t = acc_t + t * tt
                    hk.append(h)
                    tk.append(t)
                s_h = _allsum(acc_h, perms)
                s_t = _allsum(acc_t, perms)
                nh = jnp.zeros((LANES,), jnp.float32)
                nt = jnp.zeros((LANES,), jnp.float32)
                hp, tp = [], []
                for k in range(KREG):
                    rt = rtrb[i, pl.ds(k * LANES, LANES)]
                    hpk = hk[k] + s_h * rt
                    tpk = tk[k] + s_t * rt
                    nh = nh + hpk * hpk
                    nt = nt + tpk * tpk
                    hp.append(hpk)
                    tp.append(tpk)
                inv_h = _rsqrt_f32(jnp.maximum(_allsum(nh, perms), jnp.float32(1e-24)))
                inv_t = _rsqrt_f32(jnp.maximum(_allsum(nt, perms), jnp.float32(1e-24)))
                acc = jnp.zeros((LANES,), jnp.float32)
                for k in range(KREG):
                    r = rbuf[i, pl.ds(k * LANES, LANES)]
                    acc = acc + jnp.abs(hp[k] * inv_h + r - tp[k] * inv_t)
                return jnp.float32(MARGIN) - _allsum(acc, perms)

            def group_body(g, _):
                scores = jnp.zeros((LANES,), jnp.float32)
                for j in range(LANES):
                    res = tri_score(g * LANES + j)
                    scores = jnp.where(lane_iota == j, res, scores)
                outv[pl.ds(ci * CHUNK + g * LANES, LANES)] = scores
                return 0

            lax.fori_loop(0, CHUNK // LANES, group_body, 0)
            return 0

        lax.fori_loop(0, n_chunks, chunk_body, 0)
        pltpu.sync_copy(outv, out_h.at[pl.ds(base_w, per_w)])

    return sc_kernel


def kernel(hs, rs, ts, ent_embs, rel_embs, ent_transfer, rel_transfer):
    B = hs.shape[0]
    hs = hs.astype(jnp.int32)
    rs = rs.astype(jnp.int32)
    ts = ts.astype(jnp.int32)
    return _make_sc_kernel(B)(hs, rs, ts, ent_embs, rel_embs,
                              ent_transfer, rel_transfer)


# speedup vs baseline: 2.6622x; 1.0616x over previous
import functools

import jax
import jax.numpy as jnp
from jax import lax
from jax.experimental import pallas as pl
from jax.experimental.pallas import tpu as pltpu
from jax.experimental.pallas import tpu_sc as plsc

DIM = 128
MARGIN = 2.0
NC, NS = 2, 16
NW = NC * NS
CHUNK = 64
LANES = 16
KREG = DIM // LANES


def _allsum(v, perms):
    for p in perms:
        v = v + v.at[p].get(mode="promise_in_bounds")
    return v


def _rsqrt_f32(x):
    xb = lax.bitcast_convert_type(x, jnp.int32)
    yb = jnp.int32(0x5F3759DF) - lax.shift_right_logical(xb, 1)
    y = lax.bitcast_convert_type(yb, jnp.float32)
    hx = x * jnp.float32(0.5)
    for _ in range(2):
        y = y * (jnp.float32(1.5) - hx * y * y)
    return y


def _make_sc_kernel(B):
    assert B % (NW * CHUNK) == 0
    per_w = B // NW
    n_chunks = per_w // CHUNK
    assert n_chunks % 2 == 0
    mesh = plsc.VectorSubcoreMesh(core_axis_name="c", subcore_axis_name="s")
    rowbuf = pltpu.VMEM((CHUNK, DIM), jnp.float32)

    @functools.partial(
        pl.kernel,
        mesh=mesh,
        out_type=jax.ShapeDtypeStruct((B,), jnp.float32),
        scratch_types=[
            pltpu.VMEM((per_w,), jnp.int32),
            pltpu.VMEM((per_w,), jnp.int32),
            pltpu.VMEM((per_w,), jnp.int32),
            [rowbuf] * 6,
            [rowbuf] * 6,
            pltpu.VMEM((per_w,), jnp.float32),
            pltpu.SemaphoreType.DMA,
            pltpu.SemaphoreType.DMA,
        ],
    )
    def sc_kernel(hs_h, rs_h, ts_h, ee_h, re_h, et_h, rt_h, out_h,
                  hidx, ridx, tidx, set0, set1, outv, sem0, sem1):
        wid = lax.axis_index("s") * NC + lax.axis_index("c")
        base_w = wid * per_w
        lane_iota = lax.iota(jnp.int32, LANES)
        perms = [lax.bitwise_xor(lane_iota, jnp.int32(s)) for s in (8, 4, 2, 1)]

        pltpu.sync_copy(hs_h.at[pl.ds(base_w, per_w)], hidx)
        pltpu.sync_copy(rs_h.at[pl.ds(base_w, per_w)], ridx)
        pltpu.sync_copy(ts_h.at[pl.ds(base_w, per_w)], tidx)

        def copies(c, bufs, sem):
            hi = hidx.at[pl.ds(c * CHUNK, CHUNK)]
            ri = ridx.at[pl.ds(c * CHUNK, CHUNK)]
            ti = tidx.at[pl.ds(c * CHUNK, CHUNK)]
            hbuf, tbuf, rbuf, htrb, ttrb, rtrb = bufs
            return [
                pltpu.make_async_copy(ee_h.at[hi], hbuf, sem),
                pltpu.make_async_copy(ee_h.at[ti], tbuf, sem),
                pltpu.make_async_copy(re_h.at[ri], rbuf, sem),
                pltpu.make_async_copy(et_h.at[hi], htrb, sem),
                pltpu.make_async_copy(et_h.at[ti], ttrb, sem),
                pltpu.make_async_copy(rt_h.at[ri], rtrb, sem),
            ]

        def start6(c, bufs, sem):
            for cp in copies(c, bufs, sem):
                cp.start()

        def wait6(c, bufs, sem):
            for cp in copies(c, bufs, sem):
                cp.wait()

        def compute(c, bufs):
            hbuf, tbuf, rbuf, htrb, ttrb, rtrb = bufs

            def tri_score(i):
                acc_h = jnp.zeros((LANES,), jnp.float32)
                acc_t = jnp.zeros((LANES,), jnp.float32)
                hk, tk = [], []
                for k in range(KREG):
                    h = hbuf[i, pl.ds(k * LANES, LANES)]
                    ht = htrb[i, pl.ds(k * LANES, LANES)]
                    t = tbuf[i, pl.ds(k * LANES, LANES)]
                    tt = ttrb[i, pl.ds(k * LANES, LANES)]
                    acc_h = acc_h + h * ht
                    acc_t = acc_t + t * tt
                    hk.append(h)
                    tk.append(t)
                s_h = _allsum(acc_h, perms)
                s_t = _allsum(acc_t, perms)
                nh = jnp.zeros((LANES,), jnp.float32)
                nt = jnp.zeros((LANES,), jnp.float32)
                hp, tp = [], []
                for k in range(KREG):
                    rt = rtrb[i, pl.ds(k * LANES, LANES)]
                    hpk = hk[k] + s_h * rt
                    tpk = tk[k] + s_t * rt
                    nh = nh + hpk * hpk
                    nt = nt + tpk * tpk
                    hp.append(hpk)
                    tp.append(tpk)
                inv_h = _rsqrt_f32(jnp.maximum(_allsum(nh, perms), jnp.float32(1e-24)))
                inv_t = _rsqrt_f32(jnp.maximum(_allsum(nt, perms), jnp.float32(1e-24)))
                acc = jnp.zeros((LANES,), jnp.float32)
                for k in range(KREG):
                    r = rbuf[i, pl.ds(k * LANES, LANES)]
                    acc = acc + jnp.abs(hp[k] * inv_h + r - tp[k] * inv_t)
                return jnp.float32(MARGIN) - _allsum(acc, perms)

            def group_body(g, _):
                scores = jnp.zeros((LANES,), jnp.float32)
                for j in range(LANES):
                    res = tri_score(g * LANES + j)
                    scores = jnp.where(lane_iota == j, res, scores)
                outv[pl.ds(c * CHUNK + g * LANES, LANES)] = scores
                return 0

            lax.fori_loop(0, CHUNK // LANES, group_body, 0)

        start6(0, set0, sem0)

        def pair_body(i, _):
            c0 = 2 * i
            start6(c0 + 1, set1, sem1)
            wait6(c0, set0, sem0)
            compute(c0, set0)

            @pl.when(c0 + 2 < n_chunks)
            def _():
                start6(c0 + 2, set0, sem0)

            wait6(c0 + 1, set1, sem1)
            compute(c0 + 1, set1)
            return 0

        lax.fori_loop(0, n_chunks // 2, pair_body, 0)
        pltpu.sync_copy(outv, out_h.at[pl.ds(base_w, per_w)])

    return sc_kernel


def kernel(hs, rs, ts, ent_embs, rel_embs, ent_transfer, rel_transfer):
    B = hs.shape[0]
    hs = hs.astype(jnp.int32)
    rs = rs.astype(jnp.int32)
    ts = ts.astype(jnp.int32)
    return _make_sc_kernel(B)(hs, rs, ts, ent_embs, rel_embs,
                              ent_transfer, rel_transfer)


# speedup vs baseline: 3.8604x; 1.4501x over previous
import functools

import jax
import jax.numpy as jnp
from jax import lax
from jax.experimental import pallas as pl
from jax.experimental.pallas import tpu as pltpu
from jax.experimental.pallas import tpu_sc as plsc

DIM = 128
MARGIN = 2.0
NC, NS = 2, 16
NW = NC * NS
CHUNK = 64
LANES = 16
KREG = DIM // LANES


def _allsum(v, perms):
    for p in perms:
        v = v + v.at[p].get(mode="promise_in_bounds")
    return v


def _rsqrt_f32(x):
    xb = lax.bitcast_convert_type(x, jnp.int32)
    yb = jnp.int32(0x5F3759DF) - lax.shift_right_logical(xb, 1)
    y = lax.bitcast_convert_type(yb, jnp.float32)
    hx = x * jnp.float32(0.5)
    for _ in range(2):
        y = y * (jnp.float32(1.5) - hx * y * y)
    return y


def _make_sc_kernel(B):
    assert B % (NW * CHUNK) == 0
    per_w = B // NW
    n_chunks = per_w // CHUNK
    mesh = plsc.VectorSubcoreMesh(core_axis_name="c", subcore_axis_name="s")
    rowbuf = pltpu.VMEM((2, CHUNK, DIM), jnp.float32)

    @functools.partial(
        pl.kernel,
        mesh=mesh,
        out_type=jax.ShapeDtypeStruct((B,), jnp.float32),
        scratch_types=[
            pltpu.VMEM((per_w,), jnp.int32),
            pltpu.VMEM((per_w,), jnp.int32),
            pltpu.VMEM((per_w,), jnp.int32),
            [rowbuf] * 6,
            pltpu.VMEM((per_w,), jnp.float32),
            pltpu.SemaphoreType.DMA,
        ],
    )
    def sc_kernel(hs_h, rs_h, ts_h, ee_h, re_h, et_h, rt_h, out_h,
                  hidx, ridx, tidx, bufs, outv, sem):
        hbuf, tbuf, rbuf, htrb, ttrb, rtrb = bufs
        wid = lax.axis_index("s") * NC + lax.axis_index("c")
        base_w = wid * per_w
        lane_iota = lax.iota(jnp.int32, LANES)
        perms = [lax.bitwise_xor(lane_iota, jnp.int32(s)) for s in (8, 4, 2, 1)]

        pltpu.sync_copy(hs_h.at[pl.ds(base_w, per_w)], hidx)
        pltpu.sync_copy(rs_h.at[pl.ds(base_w, per_w)], ridx)
        pltpu.sync_copy(ts_h.at[pl.ds(base_w, per_w)], tidx)

        def copies(c, p):
            hi = hidx.at[pl.ds(c * CHUNK, CHUNK)]
            ri = ridx.at[pl.ds(c * CHUNK, CHUNK)]
            ti = tidx.at[pl.ds(c * CHUNK, CHUNK)]
            return [
                pltpu.make_async_copy(ee_h.at[hi], hbuf.at[p], sem),
                pltpu.make_async_copy(ee_h.at[ti], tbuf.at[p], sem),
                pltpu.make_async_copy(re_h.at[ri], rbuf.at[p], sem),
                pltpu.make_async_copy(et_h.at[hi], htrb.at[p], sem),
                pltpu.make_async_copy(et_h.at[ti], ttrb.at[p], sem),
                pltpu.make_async_copy(rt_h.at[ri], rtrb.at[p], sem),
            ]

        def start6(c, p):
            for cp in copies(c, p):
                cp.start()

        def wait6(c, p):
            for cp in copies(c, p):
                cp.wait()

        def tri_score(p, i):
            acc_h = jnp.zeros((LANES,), jnp.float32)
            acc_t = jnp.zeros((LANES,), jnp.float32)
            hk, tk = [], []
            for k in range(KREG):
                h = hbuf[p, i, pl.ds(k * LANES, LANES)]
                ht = htrb[p, i, pl.ds(k * LANES, LANES)]
                t = tbuf[p, i, pl.ds(k * LANES, LANES)]
                tt = ttrb[p, i, pl.ds(k * LANES, LANES)]
                acc_h = acc_h + h * ht
                acc_t = acc_t + t * tt
                hk.append(h)
                tk.append(t)
            s_h = _allsum(acc_h, perms)
            s_t = _allsum(acc_t, perms)
            nh = jnp.zeros((LANES,), jnp.float32)
            nt = jnp.zeros((LANES,), jnp.float32)
            hp, tp = [], []
            for k in range(KREG):
                rt = rtrb[p, i, pl.ds(k * LANES, LANES)]
                hpk = hk[k] + s_h * rt
                tpk = tk[k] + s_t * rt
                nh = nh + hpk * hpk
                nt = nt + tpk * tpk
                hp.append(hpk)
                tp.append(tpk)
            inv_h = _rsqrt_f32(jnp.maximum(_allsum(nh, perms), jnp.float32(1e-24)))
            inv_t = _rsqrt_f32(jnp.maximum(_allsum(nt, perms), jnp.float32(1e-24)))
            acc = jnp.zeros((LANES,), jnp.float32)
            for k in range(KREG):
                r = rbuf[p, i, pl.ds(k * LANES, LANES)]
                acc = acc + jnp.abs(hp[k] * inv_h + r - tp[k] * inv_t)
            return jnp.float32(MARGIN) - _allsum(acc, perms)

        start6(0, 0)

        def chunk_body(ci, _):
            p = lax.rem(ci, 2)
            wait6(ci, p)

            @pl.when(ci + 1 < n_chunks)
            def _():
                start6(ci + 1, 1 - p)

            def group_body(g, _):
                scores = jnp.zeros((LANES,), jnp.float32)
                for j in range(LANES):
                    res = tri_score(p, g * LANES + j)
                    scores = jnp.where(lane_iota == j, res, scores)
                outv[pl.ds(ci * CHUNK + g * LANES, LANES)] = scores
                return 0

            lax.fori_loop(0, CHUNK // LANES, group_body, 0)
            return 0

        lax.fori_loop(0, n_chunks, chunk_body, 0)
        pltpu.sync_copy(outv, out_h.at[pl.ds(base_w, per_w)])

    return sc_kernel


def kernel(hs, rs, ts, ent_embs, rel_embs, ent_transfer, rel_transfer):
    B = hs.shape[0]
    hs = hs.astype(jnp.int32)
    rs = rs.astype(jnp.int32)
    ts = ts.astype(jnp.int32)
    return _make_sc_kernel(B)(hs, rs, ts, ent_embs, rel_embs,
                              ent_transfer, rel_transfer)
